# baseline (speedup 1.0000x reference)
import functools

import jax
import jax.numpy as jnp
from jax import lax
from jax.experimental import pallas as pl
from jax.experimental.pallas import tpu as pltpu
from jax.experimental.pallas import tpu_sc as plsc

B = 16384
S = 6
ED = 64
ROWS = B * S
NC, NS = 2, 16
NW = NC * NS
RPW = ROWS // NW
C = 96
NCHUNK = RPW // C

_SP_N = 1600
_TY_N = 19 * 19
_IT_N = 500
_AB_N = 300

_I32 = jnp.int32
_F32 = jnp.float32


def _prep_body(sp_t, ty_t, it_t, ab_t, sw, sb, W, b,
               sp_f, typ, it_f, ab_f, stM):
    w = W[...]
    dot = functools.partial(jnp.dot, preferred_element_type=_F32,
                            precision=lax.Precision.HIGHEST)
    sp_f[...] = dot(sp_t[...], w[0:32, :])
    ty_a = dot(ty_t[...], w[32:48, :])
    ty_b = dot(ty_t[...], w[48:64, :])
    bias = b[...] + dot(sb[...], w[96:112, :])
    pair = ty_a[:, None, :] + ty_b[None, :, :] + bias[None, None, :]
    typ[...] = pair.reshape(_TY_N, ED)
    it_f[...] = dot(it_t[...], w[64:80, :])
    ab_f[...] = dot(ab_t[...], w[80:96, :])
    stM[...] = dot(sw[...], w[96:112, :])


def _prep(sp_t, ty_t, it_t, ab_t, sw, sb, W, b):
    return pl.pallas_call(
        _prep_body,
        out_shape=[
            jax.ShapeDtypeStruct((_SP_N, ED), _F32),
            jax.ShapeDtypeStruct((_TY_N, ED), _F32),
            jax.ShapeDtypeStruct((_IT_N, ED), _F32),
            jax.ShapeDtypeStruct((_AB_N, ED), _F32),
            jax.ShapeDtypeStruct((S, ED), _F32),
        ],
    )(sp_t, ty_t, it_t, ab_t, sw, sb, W, b)


def _rsqrt_vec(x):
    i = plsc.bitcast(x, _I32)
    y = plsc.bitcast(jnp.full((16,), 0x5F3759DF, _I32) - (i >> 1), _F32)
    half = x * (-0.5)
    for _ in range(2):
        y = y * (half * y * y + 1.5)
    return y


def _sc_body(sid_hbm, w_hbm, stats_hbm,
             sp_f_hbm, typ_hbm, it_hbm, ab_hbm, stM_hbm, gam_hbm, bet_hbm,
             out_hbm,
             typ_v, it_v, ab_v, stM_v, gam_v, bet_v,
             sid_v, w_v, st_v, sp0, sp1, out0, out1,
             semg0, semg1, semo0, semo1):
    wid = lax.axis_index("s") * NC + lax.axis_index("c")
    wbase = wid * RPW

    pltpu.sync_copy(typ_hbm, typ_v)
    pltpu.sync_copy(it_hbm, it_v)
    pltpu.sync_copy(ab_hbm, ab_v)
    pltpu.sync_copy(stM_hbm, stM_v)
    pltpu.sync_copy(gam_hbm, gam_v)
    pltpu.sync_copy(bet_hbm, bet_v)
    pltpu.sync_copy(sid_hbm.at[pl.ds(wbase, RPW)], sid_v)
    pltpu.sync_copy(w_hbm.at[pl.ds(wbase, RPW)], w_v)
    pltpu.sync_copy(stats_hbm.at[pl.ds(wid * (RPW * S // 128), RPW * S // 128)],
                    st_v)

    lanes = lax.iota(_I32, 16)
    cols = [lanes + 16 * j for j in range(4)]

    M = [[stM_v[pl.ds(k * ED + 16 * j, 16)] for j in range(4)]
         for k in range(S)]
    gam = [gam_v[pl.ds(16 * j, 16)] for j in range(4)]
    bet = [bet_v[pl.ds(16 * j, 16)] for j in range(4)]

    def gather_sp(c, sp_buf, sem):
        idx = sid_v.at[pl.ds(c * C, C)]
        return pltpu.async_copy(sp_f_hbm.at[idx], sp_buf, sem)

    def out_dma(c, out_buf, sem):
        bb = (wbase + c * C) // S
        return pltpu.make_async_copy(
            out_buf, out_hbm.at[pl.ds(bb, C // S)], sem)

    def _one_row(g0, rb, rs, sp_buf, out_buf):
            r = rb * S + rs
            g = jnp.full((16,), g0 + r, _I32)
            w = plsc.load_gather(w_v, [g])
            pid = (w & 511) * ED
            iid = ((w >> 9) & 511) * ED
            aid = (w >> 18) * ED
            g6 = g * S
            sk = []
            for k in range(S):
                s = g6 + k
                sk.append(plsc.load_gather(st_v, [s >> 7, s & 127]))
            h = []
            for j in range(4):
                v = sp_buf[r, pl.ds(16 * j, 16)]
                v = v + plsc.load_gather(typ_v, [pid + cols[j]])
                v = v + plsc.load_gather(it_v, [iid + cols[j]])
                v = v + plsc.load_gather(ab_v, [aid + cols[j]])
                for k in range(S):
                    v = v + sk[k] * M[k][j]
                h.append(jnp.maximum(v, 0.0))
            tot = jnp.sum((h[0] + h[1]) + (h[2] + h[3]))
            mean = tot * (1.0 / 64.0)
            mv = jnp.full((16,), mean, _F32)
            d = [h[j] - mv for j in range(4)]
            sq = jnp.sum((d[0] * d[0] + d[1] * d[1])
                         + (d[2] * d[2] + d[3] * d[3]))
            inv = _rsqrt_vec(jnp.full((16,), sq * (1.0 / 64.0) + 1e-5, _F32))
            for j in range(4):
                o = d[j] * inv * gam[j] + bet[j]
                out_buf[rb, rs, pl.ds(16 * j, 16)] = o

    def compute(c, sp_buf, out_buf):
        g0 = c * C

        @plsc.parallel_loop(0, C // S, 1, unroll=1)
        def batch_body(rb):
            for rs in range(S):
                _one_row(g0, rb, rs, sp_buf, out_buf)

    gather_sp(0, sp0, semg0)

    def pair_body(cp, _):
        c0 = 2 * cp
        c1 = c0 + 1
        gather_sp(c1, sp1, semg1)
        pltpu.make_async_copy(sp_f_hbm.at[sid_v.at[pl.ds(c0 * C, C)]],
                              sp0, semg0).wait()

        @pl.when(cp > 0)
        def _():
            out_dma(c0, out0, semo0).wait()
        compute(c0, sp0, out0)
        out_dma(c0, out0, semo0).start()

        @pl.when(cp < NCHUNK // 2 - 1)
        def _():
            gather_sp(c0 + 2, sp0, semg0)
        pltpu.make_async_copy(sp_f_hbm.at[sid_v.at[pl.ds(c1 * C, C)]],
                              sp1, semg1).wait()

        @pl.when(cp > 0)
        def _():
            out_dma(c1, out1, semo1).wait()
        compute(c1, sp1, out1)
        out_dma(c1, out1, semo1).start()
        return ()

    lax.fori_loop(0, NCHUNK // 2, pair_body, ())
    out_dma(NCHUNK - 2, out0, semo0).wait()
    out_dma(NCHUNK - 1, out1, semo1).wait()


def _sc_run(sid, w, stats_f, sp_f, typ_f, it_f, ab_f, stM_f,
            ln_gamma, ln_beta):
    mesh = plsc.VectorSubcoreMesh(core_axis_name="c", subcore_axis_name="s",
                                  num_cores=NC, num_subcores=NS)
    f = pl.kernel(
        _sc_body,
        out_type=jax.ShapeDtypeStruct((B, S, ED), _F32),
        mesh=mesh,
        compiler_params=pltpu.CompilerParams(needs_layout_passes=False,
                                             use_tc_tiling_on_sc=False),
        scratch_types=[
            pltpu.VMEM((_TY_N * ED,), _F32),
            pltpu.VMEM((_IT_N * ED,), _F32),
            pltpu.VMEM((_AB_N * ED,), _F32),
            pltpu.VMEM((S * ED,), _F32),
            pltpu.VMEM((ED,), _F32),
            pltpu.VMEM((ED,), _F32),
            pltpu.VMEM((RPW,), _I32),
            pltpu.VMEM((RPW,), _I32),
            pltpu.VMEM((RPW * S // 128, 128), _F32),
            pltpu.VMEM((C, ED), _F32),
            pltpu.VMEM((C, ED), _F32),
            pltpu.VMEM((C // S, S, ED), _F32),
            pltpu.VMEM((C // S, S, ED), _F32),
            pltpu.SemaphoreType.DMA,
            pltpu.SemaphoreType.DMA,
            pltpu.SemaphoreType.DMA,
            pltpu.SemaphoreType.DMA,
        ],
    )
    return f(sid, w, stats_f, sp_f, typ_f, it_f, ab_f, stM_f,
             ln_gamma, ln_beta)


def kernel(species_ids, type_ids, item_ids, ability_ids, stats,
           species_table, type_table, item_table, ability_table,
           stats_W, stats_b, comb_W, comb_b, ln_gamma, ln_beta):
    sp_f, typ, it_f, ab_f, stM = _prep(
        species_table, type_table, item_table, ability_table,
        stats_W, stats_b, comb_W, comb_b)
    sid = species_ids.reshape(ROWS)
    pair = type_ids[:, :, 0] * 19 + type_ids[:, :, 1]
    w = (pair | (item_ids << 9) | (ability_ids << 18)).reshape(ROWS)
    stats_f = stats.reshape(ROWS * S // 128, 128)
    return _sc_run(sid, w, stats_f,
                   sp_f, typ.reshape(_TY_N * ED), it_f.reshape(_IT_N * ED),
                   ab_f.reshape(_AB_N * ED), stM.reshape(S * ED),
                   ln_gamma, ln_beta)


# speedup vs baseline: 1.2786x; 1.2786x over previous
import functools

import jax
import jax.numpy as jnp
from jax import lax
from jax.experimental import pallas as pl
from jax.experimental.pallas import tpu as pltpu
from jax.experimental.pallas import tpu_sc as plsc

B = 16384
S = 6
ED = 64
ROWS = B * S
NC, NS = 2, 16
NW = NC * NS
RPW = ROWS // NW
C = 96
NCHUNK = RPW // C

_SP_N = 1600
_TY_N = 19 * 19
_IT_N = 500
_AB_N = 300

_I32 = jnp.int32
_F32 = jnp.float32


def _prep_body(sp_t, ty_t, it_t, ab_t, sw, sb, W, b,
               sp_f, typ, it_f, ab_f, stM):
    w = W[...]
    dot = functools.partial(jnp.dot, preferred_element_type=_F32,
                            precision=lax.Precision.HIGHEST)
    sp_f[...] = dot(sp_t[...], w[0:32, :])
    ty_a = dot(ty_t[...], w[32:48, :])
    ty_b = dot(ty_t[...], w[48:64, :])
    bias = b[...] + dot(sb[...], w[96:112, :])
    pair = ty_a[:, None, :] + ty_b[None, :, :] + bias[None, None, :]
    typ[...] = pair.reshape(_TY_N, ED)
    it_f[...] = dot(it_t[...], w[64:80, :])
    ab_f[...] = dot(ab_t[...], w[80:96, :])
    stM[...] = dot(sw[...], w[96:112, :])


def _prep(sp_t, ty_t, it_t, ab_t, sw, sb, W, b):
    return pl.pallas_call(
        _prep_body,
        out_shape=[
            jax.ShapeDtypeStruct((_SP_N, ED), _F32),
            jax.ShapeDtypeStruct((_TY_N, ED), _F32),
            jax.ShapeDtypeStruct((_IT_N, ED), _F32),
            jax.ShapeDtypeStruct((_AB_N, ED), _F32),
            jax.ShapeDtypeStruct((S, ED), _F32),
        ],
    )(sp_t, ty_t, it_t, ab_t, sw, sb, W, b)


def _rsqrt_vec(x):
    i = plsc.bitcast(x, _I32)
    y = plsc.bitcast(jnp.full((16,), 0x5F3759DF, _I32) - (i >> 1), _F32)
    half = x * (-0.5)
    for _ in range(2):
        y = y * (half * y * y + 1.5)
    return y


def _sc_body(sid_hbm, w_hbm, stats_hbm,
             sp_f_hbm, typ_hbm, it_hbm, ab_hbm, stM_hbm, gam_hbm, bet_hbm,
             out_hbm,
             typ_v, it_v, ab_v, stM_v, gam_v, bet_v,
             sid_v, w_v, st_v, sp0, sp1, out0, out1,
             semg0, semg1, semo0, semo1):
    wid = lax.axis_index("s") * NC + lax.axis_index("c")
    wbase = wid * RPW

    pltpu.sync_copy(typ_hbm, typ_v)
    pltpu.sync_copy(it_hbm, it_v)
    pltpu.sync_copy(ab_hbm, ab_v)
    pltpu.sync_copy(stM_hbm, stM_v)
    pltpu.sync_copy(gam_hbm, gam_v)
    pltpu.sync_copy(bet_hbm, bet_v)
    pltpu.sync_copy(sid_hbm.at[pl.ds(wbase, RPW)], sid_v)
    pltpu.sync_copy(w_hbm.at[pl.ds(wbase, RPW)], w_v)
    pltpu.sync_copy(stats_hbm.at[pl.ds(wid * (RPW * S // 128), RPW * S // 128)],
                    st_v)

    lanes = lax.iota(_I32, 16)
    cols = [lanes + 16 * j for j in range(4)]

    M = [[stM_v[pl.ds(k * ED + 16 * j, 16)] for j in range(4)]
         for k in range(S)]
    gam = [gam_v[pl.ds(16 * j, 16)] for j in range(4)]
    bet = [bet_v[pl.ds(16 * j, 16)] for j in range(4)]

    def gather_sp(c, sp_buf, sem):
        idx = sid_v.at[pl.ds(c * C, C)]
        return pltpu.async_copy(sp_f_hbm.at[idx], sp_buf, sem)

    def out_dma(c, out_buf, sem):
        return pltpu.make_async_copy(
            out_buf, out_hbm.at[pl.ds((wbase + c * C) * ED, C * ED)], sem)

    def compute(c, sp_buf, out_buf):
        g0 = c * C

        @plsc.parallel_loop(0, C, 1, unroll=2)
        def row_body(r):
            g = jnp.full((16,), g0 + r, _I32)
            w = plsc.load_gather(w_v, [g])
            pid = (w & 511) * ED
            iid = ((w >> 9) & 511) * ED
            aid = (w >> 18) * ED
            g6 = g * S
            sk = []
            for k in range(S):
                s = g6 + k
                sk.append(plsc.load_gather(st_v, [s >> 7, s & 127]))
            h = []
            for j in range(4):
                v = sp_buf[r, pl.ds(16 * j, 16)]
                v = v + plsc.load_gather(typ_v, [pid + cols[j]])
                v = v + plsc.load_gather(it_v, [iid + cols[j]])
                v = v + plsc.load_gather(ab_v, [aid + cols[j]])
                for k in range(S):
                    v = v + sk[k] * M[k][j]
                h.append(jnp.maximum(v, 0.0))
            tot = jnp.sum((h[0] + h[1]) + (h[2] + h[3]))
            mean = tot * (1.0 / 64.0)
            mv = jnp.full((16,), mean, _F32)
            d = [h[j] - mv for j in range(4)]
            sq = jnp.sum((d[0] * d[0] + d[1] * d[1])
                         + (d[2] * d[2] + d[3] * d[3]))
            inv = _rsqrt_vec(jnp.full((16,), sq * (1.0 / 64.0) + 1e-5, _F32))
            r64 = r * ED
            for j in range(4):
                o = d[j] * inv * gam[j] + bet[j]
                out_buf[pl.ds(r64 + 16 * j, 16)] = o

    gather_sp(0, sp0, semg0)

    def pair_body(cp, _):
        c0 = 2 * cp
        c1 = c0 + 1
        gather_sp(c1, sp1, semg1)
        pltpu.make_async_copy(sp_f_hbm.at[sid_v.at[pl.ds(c0 * C, C)]],
                              sp0, semg0).wait()

        @pl.when(cp > 0)
        def _():
            out_dma(c0, out0, semo0).wait()
        compute(c0, sp0, out0)
        out_dma(c0, out0, semo0).start()

        @pl.when(cp < NCHUNK // 2 - 1)
        def _():
            gather_sp(c0 + 2, sp0, semg0)
        pltpu.make_async_copy(sp_f_hbm.at[sid_v.at[pl.ds(c1 * C, C)]],
                              sp1, semg1).wait()

        @pl.when(cp > 0)
        def _():
            out_dma(c1, out1, semo1).wait()
        compute(c1, sp1, out1)
        out_dma(c1, out1, semo1).start()
        return ()

    lax.fori_loop(0, NCHUNK // 2, pair_body, ())
    out_dma(NCHUNK - 2, out0, semo0).wait()
    out_dma(NCHUNK - 1, out1, semo1).wait()


def _sc_run(sid, w, stats_f, sp_f, typ_f, it_f, ab_f, stM_f,
            ln_gamma, ln_beta):
    mesh = plsc.VectorSubcoreMesh(core_axis_name="c", subcore_axis_name="s",
                                  num_cores=NC, num_subcores=NS)
    f = pl.kernel(
        _sc_body,
        out_type=jax.ShapeDtypeStruct((ROWS * ED,), _F32),
        mesh=mesh,
        compiler_params=pltpu.CompilerParams(needs_layout_passes=False,
                                             use_tc_tiling_on_sc=False),
        scratch_types=[
            pltpu.VMEM((_TY_N * ED,), _F32),
            pltpu.VMEM((_IT_N * ED,), _F32),
            pltpu.VMEM((_AB_N * ED,), _F32),
            pltpu.VMEM((S * ED,), _F32),
            pltpu.VMEM((ED,), _F32),
            pltpu.VMEM((ED,), _F32),
            pltpu.VMEM((RPW,), _I32),
            pltpu.VMEM((RPW,), _I32),
            pltpu.VMEM((RPW * S // 128, 128), _F32),
            pltpu.VMEM((C, ED), _F32),
            pltpu.VMEM((C, ED), _F32),
            pltpu.VMEM((C * ED,), _F32),
            pltpu.VMEM((C * ED,), _F32),
            pltpu.SemaphoreType.DMA,
            pltpu.SemaphoreType.DMA,
            pltpu.SemaphoreType.DMA,
            pltpu.SemaphoreType.DMA,
        ],
    )
    return f(sid, w, stats_f, sp_f, typ_f, it_f, ab_f, stM_f,
             ln_gamma, ln_beta)


def kernel(species_ids, type_ids, item_ids, ability_ids, stats,
           species_table, type_table, item_table, ability_table,
           stats_W, stats_b, comb_W, comb_b, ln_gamma, ln_beta):
    sp_f, typ, it_f, ab_f, stM = _prep(
        species_table, type_table, item_table, ability_table,
        stats_W, stats_b, comb_W, comb_b)
    sid = species_ids.reshape(ROWS)
    pair = type_ids[:, :, 0] * 19 + type_ids[:, :, 1]
    w = (pair | (item_ids << 9) | (ability_ids << 18)).reshape(ROWS)
    stats_f = stats.reshape(ROWS * S // 128, 128)
    out = _sc_run(sid, w, stats_f,
                  sp_f, typ.reshape(_TY_N * ED), it_f.reshape(_IT_N * ED),
                  ab_f.reshape(_AB_N * ED), stM.reshape(S * ED),
                  ln_gamma, ln_beta)
    return out.reshape(B, S, ED)
